# Initial kernel scaffold; baseline (speedup 1.0000x reference)
import jax
import jax.numpy as jnp
from jax.experimental import pallas as pl


def kernel(x, edge_index, W, a_l, a_r):
    raise NotImplementedError("write your pallas kernel here")


# speedup vs baseline: 10.4894x; 10.4894x over previous
import dataclasses
import functools

import jax
import jax.numpy as jnp
from jax import lax
from jax.experimental import pallas as pl
from jax.experimental.pallas import tpu as pltpu
from jax.experimental.pallas import tpu_sc as plsc

KBLK = 80
NSUB = 16
NCORE = 2
NPAD = 10240


def _proj_body(x_ref, w_ref, al_ref, ar_ref, h_ref, el_ref, er_ref):
    h = jnp.dot(x_ref[...], w_ref[...], preferred_element_type=jnp.float32)
    h_ref[...] = h
    el_ref[...] = jnp.dot(h, al_ref[...], preferred_element_type=jnp.float32)
    er_ref[...] = jnp.dot(h, ar_ref[...], preferred_element_type=jnp.float32)


def _final_body(q0_ref, q1_ref, s_ref, x_ref, o_ref):
    s = s_ref[...] + 1e-16
    num = jnp.concatenate([q0_ref[...], q1_ref[...]], axis=1)
    o_ref[...] = num / s + x_ref[...]


def _sc_edge_kernel(n_nodes, n_edges):
    epw = n_edges // NSUB
    nrows = n_nodes // 2
    mrs = 312
    rem = nrows - NSUB * mrs
    spn = NPAD // NSUB
    mesh = plsc.VectorSubcoreMesh(core_axis_name="c", subcore_axis_name="s")
    cp = pltpu.CompilerParams()
    if "needs_layout_passes" in pltpu.CompilerParams.__dataclass_fields__:
        cp = dataclasses.replace(cp, needs_layout_passes=False)

    @functools.partial(
        pl.kernel,
        mesh=mesh,
        compiler_params=cp,
        out_type=(
            jax.ShapeDtypeStruct((NCORE, nrows, 128), jnp.float32),
            jax.ShapeDtypeStruct((NPAD,), jnp.float32),
            jax.ShapeDtypeStruct((NSUB * NPAD,), jnp.float32),
        ),
        scratch_types=[
            pltpu.VMEM((n_nodes,), jnp.float32),
            pltpu.VMEM((n_nodes,), jnp.float32),
            pltpu.VMEM((KBLK,), jnp.int32),
            pltpu.VMEM((KBLK,), jnp.int32),
            pltpu.VMEM((KBLK,), jnp.int32),
            pltpu.VMEM((KBLK, 128), jnp.float32),
            pltpu.VMEM((KBLK, 128), jnp.float32),
            pltpu.VMEM((NPAD,), jnp.float32),
            pltpu.VMEM((NPAD,), jnp.float32),
            pltpu.VMEM((spn,), jnp.float32),
            pltpu.VMEM_SHARED((nrows, 128), jnp.float32),
            pltpu.SemaphoreType.DMA,
        ],
    )
    def edge_kernel(h_hbm, src_hbm, dst_hbm, el_hbm, er_hbm, z_hbm,
                    outp_hbm, outs_hbm, sp_hbm,
                    el_v, er_v, srcb, dstb, rowb, rows, msg,
                    s_loc, s_blk, s_out, acc_sh, sem):
        cid = lax.axis_index("c")
        sid = lax.axis_index("s")
        pltpu.sync_copy(el_hbm, el_v)
        pltpu.sync_copy(er_hbm, er_v)
        zvec = jnp.zeros((16,), jnp.float32)

        @pl.loop(0, NPAD, step=16)
        def _(i):
            s_loc[pl.ds(i, 16)] = zvec

        pltpu.sync_copy(z_hbm.at[pl.ds(sid * mrs, mrs)],
                        acc_sh.at[pl.ds(sid * mrs, mrs)])

        @pl.when(sid == 0)
        def _():
            pltpu.sync_copy(z_hbm.at[pl.ds(NSUB * mrs, rem)],
                            acc_sh.at[pl.ds(NSUB * mrs, rem)])
        plsc.subcore_barrier()

        lane = lax.iota(jnp.int32, 16)
        masks = [lane == jl for jl in range(16)]
        base_e = sid * epw
        is_c0 = cid == 0

        @pl.loop(0, epw, step=KBLK)
        def _(b):
            pltpu.sync_copy(src_hbm.at[pl.ds(base_e + b, KBLK)], srcb)
            pltpu.sync_copy(dst_hbm.at[pl.ds(base_e + b, KBLK)], dstb)
            pltpu.async_copy(h_hbm.at[srcb], rows, sem).wait()

            for c16 in range(KBLK // 16):
                sv = srcb[pl.ds(c16 * 16, 16)]
                dv = dstb[pl.ds(c16 * 16, 16)]
                rowb[pl.ds(c16 * 16, 16)] = lax.shift_right_logical(dv, 1)
                parf_vec = (dv & 1).astype(jnp.float32)
                e = plsc.load_gather(el_v, [sv]) + plsc.load_gather(er_v, [dv])
                e = jnp.where(e > 0, e, 0.2 * e)
                w_vec = jnp.exp(e)

                @pl.when(is_c0)
                def _():
                    for jl in range(16):
                        plsc.addupdate_scatter(s_loc, [dv], w_vec,
                                               mask=masks[jl])
                    for jl in range(16):
                        j = c16 * 16 + jl
                        w_hi = w_vec[jl] * parf_vec[jl]
                        w_lo = w_vec[jl] - w_hi
                        for k in range(4):
                            v = rows[j, pl.ds(k * 16, 16)]
                            msg[j, pl.ds(k * 16, 16)] = v * w_lo
                            msg[j, pl.ds(64 + k * 16, 16)] = v * w_hi

                @pl.when(jnp.logical_not(is_c0))
                def _():
                    for jl in range(16):
                        j = c16 * 16 + jl
                        w_hi = w_vec[jl] * parf_vec[jl]
                        w_lo = w_vec[jl] - w_hi
                        for k in range(4):
                            v = rows[j, pl.ds(64 + k * 16, 16)]
                            msg[j, pl.ds(k * 16, 16)] = v * w_lo
                            msg[j, pl.ds(64 + k * 16, 16)] = v * w_hi

            pltpu.sync_copy(msg, acc_sh.at[rowb], add=True)

        @pl.when(is_c0)
        def _():
            pltpu.sync_copy(s_loc, sp_hbm.at[pl.ds(sid * NPAD, NPAD)])
        plsc.subcore_barrier()

        @pl.when(is_c0)
        def _():
            for t in range(NSUB):
                pltpu.sync_copy(sp_hbm.at[pl.ds(t * NPAD + sid * spn, spn)],
                                s_blk.at[pl.ds(t * spn, spn)])

            @pl.loop(0, spn, step=16)
            def _(g):
                tot = s_blk[pl.ds(g, 16)]
                for t in range(1, NSUB):
                    tot = tot + s_blk[pl.ds(t * spn + g, 16)]
                s_out[pl.ds(g, 16)] = tot

            pltpu.sync_copy(s_out, outs_hbm.at[pl.ds(sid * spn, spn)])

        pltpu.sync_copy(
            acc_sh.at[pl.ds(sid * mrs, mrs)],
            outp_hbm.at[cid, pl.ds(sid * mrs, mrs)])

        @pl.when(sid == 0)
        def _():
            pltpu.sync_copy(
                acc_sh.at[pl.ds(NSUB * mrs, rem)],
                outp_hbm.at[cid, pl.ds(NSUB * mrs, rem)])

    return edge_kernel


def kernel(x, edge_index, W, a_l, a_r):
    n, d = x.shape
    e = edge_index.shape[1]

    h, el2, er2 = pl.pallas_call(
        _proj_body,
        out_shape=(
            jax.ShapeDtypeStruct((n, d), jnp.float32),
            jax.ShapeDtypeStruct((n, 1), jnp.float32),
            jax.ShapeDtypeStruct((n, 1), jnp.float32),
        ),
    )(x, W, a_l.reshape(d, 1), a_r.reshape(d, 1))

    zacc = jnp.zeros((n // 2, 128), jnp.float32)
    partials, s, _ = _sc_edge_kernel(n, e)(
        h, edge_index[0], edge_index[1],
        el2.reshape(n), er2.reshape(n), zacc)

    out = pl.pallas_call(
        _final_body,
        out_shape=jax.ShapeDtypeStruct((n, d), jnp.float32),
    )(partials[0].reshape(n, 64), partials[1].reshape(n, 64),
      s[:n].reshape(n, 1), x)
    return out


# speedup vs baseline: 19.8885x; 1.8961x over previous
import dataclasses
import functools

import jax
import jax.numpy as jnp
from jax import lax
from jax.experimental import pallas as pl
from jax.experimental.pallas import tpu as pltpu
from jax.experimental.pallas import tpu_sc as plsc

KBLK = 80
SBE = 4000
NSUB = 16
NCORE = 2
NPAD = 10240


def _proj_body(x_ref, w_ref, al_ref, ar_ref, h0_ref, h1_ref, el_ref, er_ref):
    h = jnp.dot(x_ref[...], w_ref[...], preferred_element_type=jnp.float32)
    h0_ref[...] = h
    h1_ref[...] = jnp.concatenate([h[:, 64:], h[:, :64]], axis=1)
    el_ref[...] = jnp.dot(h, al_ref[...], preferred_element_type=jnp.float32)
    er_ref[...] = jnp.dot(h, ar_ref[...], preferred_element_type=jnp.float32)


def _final_body(q0_ref, q1_ref, s_ref, x_ref, o_ref):
    s = s_ref[...] + 1e-16
    num = jnp.concatenate([q0_ref[...], q1_ref[...]], axis=1)
    o_ref[...] = num / s + x_ref[...]


def _sc_edge_kernel(n_nodes, n_edges):
    epw = n_edges // NSUB
    nrows = n_nodes // 2
    mrs = 312
    rem = nrows - NSUB * mrs
    spn = NPAD // NSUB
    mesh = plsc.VectorSubcoreMesh(core_axis_name="c", subcore_axis_name="s")
    cp = pltpu.CompilerParams()
    if "needs_layout_passes" in pltpu.CompilerParams.__dataclass_fields__:
        cp = dataclasses.replace(cp, needs_layout_passes=False)

    @functools.partial(
        pl.kernel,
        mesh=mesh,
        compiler_params=cp,
        out_type=(
            jax.ShapeDtypeStruct((NCORE, nrows, 128), jnp.float32),
            jax.ShapeDtypeStruct((NPAD,), jnp.float32),
            jax.ShapeDtypeStruct((NSUB * NPAD,), jnp.float32),
        ),
        scratch_types=[
            pltpu.VMEM((n_nodes,), jnp.float32),
            pltpu.VMEM((n_nodes,), jnp.float32),
            pltpu.VMEM((SBE,), jnp.int32),
            pltpu.VMEM((SBE,), jnp.int32),
            pltpu.VMEM((KBLK,), jnp.int32),
            pltpu.VMEM((KBLK,), jnp.int32),
            pltpu.VMEM((KBLK, 128), jnp.float32),
            pltpu.VMEM((KBLK, 128), jnp.float32),
            pltpu.VMEM((KBLK, 128), jnp.float32),
            pltpu.VMEM((KBLK, 128), jnp.float32),
            pltpu.VMEM((NPAD,), jnp.float32),
            pltpu.VMEM((spn,), jnp.float32),
            pltpu.VMEM_SHARED((nrows, 128), jnp.float32),
            pltpu.SemaphoreType.DMA,
            pltpu.SemaphoreType.DMA,
            pltpu.SemaphoreType.DMA,
            pltpu.SemaphoreType.DMA,
            pltpu.SemaphoreType.DMA,
        ],
    )
    def edge_kernel(h0_hbm, h1_hbm, src_hbm, dst_hbm, el_hbm, er_hbm, z_hbm,
                    outp_hbm, outs_hbm, sp_hbm,
                    el_v, er_v, srca, dsta, rowb_a, rowb_b,
                    rows_a, rows_b, msg_a, msg_b,
                    s_loc, s_out, acc_sh,
                    sem, sem_ga, sem_gb, sem_sa, sem_sb):
        cid = lax.axis_index("c")
        sid = lax.axis_index("s")
        pltpu.sync_copy(el_hbm, el_v)
        pltpu.sync_copy(er_hbm, er_v)
        zvec = jnp.zeros((16,), jnp.float32)

        @pl.loop(0, NPAD, step=16)
        def _(i):
            s_loc[pl.ds(i, 16)] = zvec

        pltpu.sync_copy(z_hbm.at[pl.ds(sid * mrs, mrs)],
                        acc_sh.at[pl.ds(sid * mrs, mrs)])

        @pl.when(sid == 0)
        def _():
            pltpu.sync_copy(z_hbm.at[pl.ds(NSUB * mrs, rem)],
                            acc_sh.at[pl.ds(NSUB * mrs, rem)])
        plsc.subcore_barrier()

        lane = lax.iota(jnp.int32, 16)
        masks = [lane == jl for jl in range(16)]
        base_e = sid * epw
        is_c0 = cid == 0
        npair = SBE // (2 * KBLK)

        def gather_start(b, rows_x, sem_x):
            @pl.when(is_c0)
            def _():
                pltpu.async_copy(
                    h0_hbm.at[srca.at[pl.ds(b, KBLK)]], rows_x, sem_x)

            @pl.when(jnp.logical_not(is_c0))
            def _():
                pltpu.async_copy(
                    h1_hbm.at[srca.at[pl.ds(b, KBLK)]], rows_x, sem_x)

        def gather_wait(b, rows_x, sem_x):
            pltpu.make_async_copy(
                h0_hbm.at[srca.at[pl.ds(b, KBLK)]], rows_x, sem_x).wait()

        def scatter_start(msg_x, rowb_x, sem_x):
            pltpu.async_copy(msg_x, acc_sh.at[rowb_x], sem_x, add=True)

        def scatter_wait(msg_x, rowb_x, sem_x):
            pltpu.make_async_copy(msg_x, acc_sh.at[rowb_x], sem_x).wait()

        def process(b, rows_x, msg_x, rowb_x):
            for c16 in range(KBLK // 16):
                sv = srca[pl.ds(b + c16 * 16, 16)]
                dv = dsta[pl.ds(b + c16 * 16, 16)]
                rowb_x[pl.ds(c16 * 16, 16)] = lax.shift_right_logical(dv, 1)
                parf_vec = (dv & 1).astype(jnp.float32)
                e = plsc.load_gather(el_v, [sv]) + plsc.load_gather(er_v, [dv])
                e = jnp.where(e > 0, e, 0.2 * e)
                w_vec = jnp.exp(e)

                @pl.when(is_c0)
                def _():
                    for jl in range(16):
                        plsc.addupdate_scatter(s_loc, [dv], w_vec,
                                               mask=masks[jl])
                for jl in range(16):
                    j = c16 * 16 + jl
                    w_hi = w_vec[jl] * parf_vec[jl]
                    w_lo = w_vec[jl] - w_hi
                    for k in range(4):
                        v = rows_x[j, pl.ds(k * 16, 16)]
                        msg_x[j, pl.ds(k * 16, 16)] = v * w_lo
                        msg_x[j, pl.ds(64 + k * 16, 16)] = v * w_hi

        @pl.loop(0, epw, step=SBE)
        def _(sb):
            pltpu.sync_copy(src_hbm.at[pl.ds(base_e + sb, SBE)], srca)
            pltpu.sync_copy(dst_hbm.at[pl.ds(base_e + sb, SBE)], dsta)
            gather_start(0, rows_a, sem_ga)

            @pl.loop(0, npair)
            def _(t):
                b0 = 2 * KBLK * t
                b1 = b0 + KBLK
                gather_start(b1, rows_b, sem_gb)

                @pl.when(t > 0)
                def _():
                    scatter_wait(msg_a, rowb_a, sem_sa)
                gather_wait(b0, rows_a, sem_ga)
                process(b0, rows_a, msg_a, rowb_a)
                scatter_start(msg_a, rowb_a, sem_sa)

                @pl.when(t < npair - 1)
                def _():
                    gather_start(b0 + 2 * KBLK, rows_a, sem_ga)

                @pl.when(t > 0)
                def _():
                    scatter_wait(msg_b, rowb_b, sem_sb)
                gather_wait(b1, rows_b, sem_gb)
                process(b1, rows_b, msg_b, rowb_b)
                scatter_start(msg_b, rowb_b, sem_sb)

            scatter_wait(msg_a, rowb_a, sem_sa)
            scatter_wait(msg_b, rowb_b, sem_sb)

        @pl.when(is_c0)
        def _():
            pltpu.sync_copy(s_loc, sp_hbm.at[pl.ds(sid * NPAD, NPAD)])
        plsc.subcore_barrier()

        @pl.when(is_c0)
        def _():
            for t in range(NSUB):
                pltpu.sync_copy(sp_hbm.at[pl.ds(t * NPAD + sid * spn, spn)],
                                s_loc.at[pl.ds(t * spn, spn)])

            @pl.loop(0, spn, step=16)
            def _(g):
                tot = s_loc[pl.ds(g, 16)]
                for t in range(1, NSUB):
                    tot = tot + s_loc[pl.ds(t * spn + g, 16)]
                s_out[pl.ds(g, 16)] = tot

            pltpu.sync_copy(s_out, outs_hbm.at[pl.ds(sid * spn, spn)])

        pltpu.sync_copy(
            acc_sh.at[pl.ds(sid * mrs, mrs)],
            outp_hbm.at[cid, pl.ds(sid * mrs, mrs)])

        @pl.when(sid == 0)
        def _():
            pltpu.sync_copy(
                acc_sh.at[pl.ds(NSUB * mrs, rem)],
                outp_hbm.at[cid, pl.ds(NSUB * mrs, rem)])

    return edge_kernel


def kernel(x, edge_index, W, a_l, a_r):
    n, d = x.shape
    e = edge_index.shape[1]

    h0, h1, el2, er2 = pl.pallas_call(
        _proj_body,
        out_shape=(
            jax.ShapeDtypeStruct((n, d), jnp.float32),
            jax.ShapeDtypeStruct((n, d), jnp.float32),
            jax.ShapeDtypeStruct((n, 1), jnp.float32),
            jax.ShapeDtypeStruct((n, 1), jnp.float32),
        ),
    )(x, W, a_l.reshape(d, 1), a_r.reshape(d, 1))

    zacc = jnp.zeros((n // 2, 128), jnp.float32)
    partials, s, _ = _sc_edge_kernel(n, e)(
        h0, h1, edge_index[0], edge_index[1],
        el2.reshape(n), er2.reshape(n), zacc)

    out = pl.pallas_call(
        _final_body,
        out_shape=jax.ShapeDtypeStruct((n, d), jnp.float32),
    )(partials[0].reshape(n, 64), partials[1].reshape(n, 64),
      s[:n].reshape(n, 1), x)
    return out


# speedup vs baseline: 20.4486x; 1.0282x over previous
import dataclasses
import functools

import jax
import jax.numpy as jnp
from jax import lax
from jax.experimental import pallas as pl
from jax.experimental.pallas import tpu as pltpu
from jax.experimental.pallas import tpu_sc as plsc

KBLK = 80
SBE = 4000
NSUB = 16
NCORE = 2
NPAD = 10240


def _proj_body(x_ref, w_ref, al_ref, ar_ref, h0_ref, h1_ref, el_ref, er_ref):
    h = jnp.dot(x_ref[...], w_ref[...], preferred_element_type=jnp.float32)
    h0_ref[...] = h
    h1_ref[...] = jnp.concatenate([h[:, 64:], h[:, :64]], axis=1)
    el_ref[...] = jnp.dot(h, al_ref[...], preferred_element_type=jnp.float32)
    er_ref[...] = jnp.dot(h, ar_ref[...], preferred_element_type=jnp.float32)


def _final_body(q0_ref, q1_ref, s_ref, x_ref, o_ref):
    s = s_ref[...] + 1e-16
    num = jnp.concatenate([q0_ref[...], q1_ref[...]], axis=1)
    o_ref[...] = num / s + x_ref[...]


def _sc_edge_kernel(n_nodes, n_edges):
    epw = n_edges // NSUB
    nrows = n_nodes // 2
    mrs = 312
    rem = nrows - NSUB * mrs
    spn = NPAD // NSUB
    mesh = plsc.VectorSubcoreMesh(core_axis_name="c", subcore_axis_name="s")
    cp = pltpu.CompilerParams()
    if "needs_layout_passes" in pltpu.CompilerParams.__dataclass_fields__:
        cp = dataclasses.replace(cp, needs_layout_passes=False)

    @functools.partial(
        pl.kernel,
        mesh=mesh,
        compiler_params=cp,
        out_type=(
            jax.ShapeDtypeStruct((NCORE, nrows, 128), jnp.float32),
            jax.ShapeDtypeStruct((NPAD,), jnp.float32),
            jax.ShapeDtypeStruct((NSUB * NPAD,), jnp.float32),
        ),
        scratch_types=[
            pltpu.VMEM((n_nodes,), jnp.float32),
            pltpu.VMEM((n_nodes,), jnp.float32),
            pltpu.VMEM((SBE,), jnp.int32),
            pltpu.VMEM((SBE,), jnp.int32),
            pltpu.VMEM((KBLK,), jnp.int32),
            pltpu.VMEM((KBLK,), jnp.int32),
            pltpu.VMEM((KBLK, 128), jnp.float32),
            pltpu.VMEM((KBLK, 128), jnp.float32),
            pltpu.VMEM((KBLK, 128), jnp.float32),
            pltpu.VMEM((KBLK, 128), jnp.float32),
            pltpu.VMEM((NPAD,), jnp.float32),
            pltpu.VMEM((spn,), jnp.float32),
            pltpu.VMEM_SHARED((nrows, 128), jnp.float32),
            pltpu.SemaphoreType.DMA,
            pltpu.SemaphoreType.DMA,
            pltpu.SemaphoreType.DMA,
            pltpu.SemaphoreType.DMA,
            pltpu.SemaphoreType.DMA,
        ],
    )
    def edge_kernel(h0_hbm, h1_hbm, src_hbm, dst_hbm, el_hbm, er_hbm, z_hbm,
                    outp_hbm, outs_hbm, sp_hbm,
                    el_v, er_v, srca, dsta, rowb_a, rowb_b,
                    rows_a, rows_b, msg_a, msg_b,
                    s_loc, s_out, acc_sh,
                    sem, sem_ga, sem_gb, sem_sa, sem_sb):
        cid = lax.axis_index("c")
        sid = lax.axis_index("s")
        pltpu.sync_copy(el_hbm, el_v)
        pltpu.sync_copy(er_hbm, er_v)
        zvec = jnp.zeros((16,), jnp.float32)

        @pl.loop(0, NPAD, step=16)
        def _(i):
            s_loc[pl.ds(i, 16)] = zvec

        pltpu.sync_copy(z_hbm.at[pl.ds(sid * mrs, mrs)],
                        acc_sh.at[pl.ds(sid * mrs, mrs)])

        @pl.when(sid == 0)
        def _():
            pltpu.sync_copy(z_hbm.at[pl.ds(NSUB * mrs, rem)],
                            acc_sh.at[pl.ds(NSUB * mrs, rem)])
        plsc.subcore_barrier()

        lane = lax.iota(jnp.int32, 16)
        masks = [lane == jl for jl in range(16)]
        base_e = sid * epw
        is_c0 = cid == 0
        npair = SBE // (2 * KBLK)

        def gather_start(b, rows_x, sem_x):
            @pl.when(is_c0)
            def _():
                pltpu.async_copy(
                    h0_hbm.at[srca.at[pl.ds(b, KBLK)]], rows_x, sem_x)

            @pl.when(jnp.logical_not(is_c0))
            def _():
                pltpu.async_copy(
                    h1_hbm.at[srca.at[pl.ds(b, KBLK)]], rows_x, sem_x)

        def gather_wait(b, rows_x, sem_x):
            pltpu.make_async_copy(
                h0_hbm.at[srca.at[pl.ds(b, KBLK)]], rows_x, sem_x).wait()

        def scatter_start(msg_x, rowb_x, sem_x):
            pltpu.async_copy(msg_x, acc_sh.at[rowb_x], sem_x, add=True)

        def scatter_wait(msg_x, rowb_x, sem_x):
            pltpu.make_async_copy(msg_x, acc_sh.at[rowb_x], sem_x).wait()

        def process(b, rows_x, msg_x, rowb_x):
            for c16 in range(KBLK // 16):
                sv = srca[pl.ds(b + c16 * 16, 16)]
                dv = dsta[pl.ds(b + c16 * 16, 16)]
                rowb_x[pl.ds(c16 * 16, 16)] = lax.shift_right_logical(dv, 1)
                parf_vec = (dv & 1).astype(jnp.float32)
                e = plsc.load_gather(el_v, [sv]) + plsc.load_gather(er_v, [dv])
                e = jnp.where(e > 0, e, 0.2 * e)
                w_vec = jnp.exp(e)

                @pl.when(is_c0)
                def _():
                    plsc.addupdate_scatter(s_loc, [dv], w_vec)
                for jl in range(16):
                    j = c16 * 16 + jl
                    w_hi = w_vec[jl] * parf_vec[jl]
                    w_lo = w_vec[jl] - w_hi
                    for k in range(4):
                        v = rows_x[j, pl.ds(k * 16, 16)]
                        msg_x[j, pl.ds(k * 16, 16)] = v * w_lo
                        msg_x[j, pl.ds(64 + k * 16, 16)] = v * w_hi

        @pl.loop(0, epw, step=SBE)
        def _(sb):
            pltpu.sync_copy(src_hbm.at[pl.ds(base_e + sb, SBE)], srca)
            pltpu.sync_copy(dst_hbm.at[pl.ds(base_e + sb, SBE)], dsta)
            gather_start(0, rows_a, sem_ga)

            @pl.loop(0, npair)
            def _(t):
                b0 = 2 * KBLK * t
                b1 = b0 + KBLK
                gather_start(b1, rows_b, sem_gb)

                @pl.when(t > 0)
                def _():
                    scatter_wait(msg_a, rowb_a, sem_sa)
                gather_wait(b0, rows_a, sem_ga)
                process(b0, rows_a, msg_a, rowb_a)
                scatter_start(msg_a, rowb_a, sem_sa)

                @pl.when(t < npair - 1)
                def _():
                    gather_start(b0 + 2 * KBLK, rows_a, sem_ga)

                @pl.when(t > 0)
                def _():
                    scatter_wait(msg_b, rowb_b, sem_sb)
                gather_wait(b1, rows_b, sem_gb)
                process(b1, rows_b, msg_b, rowb_b)
                scatter_start(msg_b, rowb_b, sem_sb)

            scatter_wait(msg_a, rowb_a, sem_sa)
            scatter_wait(msg_b, rowb_b, sem_sb)

        @pl.when(is_c0)
        def _():
            pltpu.sync_copy(s_loc, sp_hbm.at[pl.ds(sid * NPAD, NPAD)])
        plsc.subcore_barrier()

        @pl.when(is_c0)
        def _():
            for t in range(NSUB):
                pltpu.sync_copy(sp_hbm.at[pl.ds(t * NPAD + sid * spn, spn)],
                                s_loc.at[pl.ds(t * spn, spn)])

            @pl.loop(0, spn, step=16)
            def _(g):
                tot = s_loc[pl.ds(g, 16)]
                for t in range(1, NSUB):
                    tot = tot + s_loc[pl.ds(t * spn + g, 16)]
                s_out[pl.ds(g, 16)] = tot

            pltpu.sync_copy(s_out, outs_hbm.at[pl.ds(sid * spn, spn)])

        pltpu.sync_copy(
            acc_sh.at[pl.ds(sid * mrs, mrs)],
            outp_hbm.at[cid, pl.ds(sid * mrs, mrs)])

        @pl.when(sid == 0)
        def _():
            pltpu.sync_copy(
                acc_sh.at[pl.ds(NSUB * mrs, rem)],
                outp_hbm.at[cid, pl.ds(NSUB * mrs, rem)])

    return edge_kernel


def kernel(x, edge_index, W, a_l, a_r):
    n, d = x.shape
    e = edge_index.shape[1]

    h0, h1, el2, er2 = pl.pallas_call(
        _proj_body,
        out_shape=(
            jax.ShapeDtypeStruct((n, d), jnp.float32),
            jax.ShapeDtypeStruct((n, d), jnp.float32),
            jax.ShapeDtypeStruct((n, 1), jnp.float32),
            jax.ShapeDtypeStruct((n, 1), jnp.float32),
        ),
    )(x, W, a_l.reshape(d, 1), a_r.reshape(d, 1))

    zacc = jnp.zeros((n // 2, 128), jnp.float32)
    partials, s, _ = _sc_edge_kernel(n, e)(
        h0, h1, edge_index[0], edge_index[1],
        el2.reshape(n), er2.reshape(n), zacc)

    out = pl.pallas_call(
        _final_body,
        out_shape=jax.ShapeDtypeStruct((n, d), jnp.float32),
    )(partials[0].reshape(n, 64), partials[1].reshape(n, 64),
      s[:n].reshape(n, 1), x)
    return out


# speedup vs baseline: 33.4869x; 1.6376x over previous
import dataclasses
import functools

import jax
import jax.numpy as jnp
from jax import lax
from jax.experimental import pallas as pl
from jax.experimental.pallas import tpu as pltpu
from jax.experimental.pallas import tpu_sc as plsc

KBLK = 80
SBE = 2000
SBW = 2000
NSUB = 16
NCORE = 2
NPAD = 10240


def _proj_body(x_ref, w_ref, al_ref, ar_ref, h_ref, el_ref, er_ref):
    h = jnp.dot(x_ref[...], w_ref[...], preferred_element_type=jnp.float32)
    h_ref[...] = h
    el_ref[...] = jnp.dot(h, al_ref[...], preferred_element_type=jnp.float32)
    er_ref[...] = jnp.dot(h, ar_ref[...], preferred_element_type=jnp.float32)


def _final_body(p_ref, s0_ref, s1_ref, x_ref, o_ref):
    s = s0_ref[...] + s1_ref[...] + 1e-16
    o_ref[...] = (p_ref[0] + p_ref[1]) / s + x_ref[...]


def _mesh_and_params():
    mesh = plsc.VectorSubcoreMesh(core_axis_name="c", subcore_axis_name="s")
    cp = pltpu.CompilerParams()
    if "needs_layout_passes" in pltpu.CompilerParams.__dataclass_fields__:
        cp = dataclasses.replace(cp, needs_layout_passes=False)
    return mesh, cp


def _sc_weight_kernel(n_nodes, n_edges):
    epw = n_edges // (NSUB * NCORE)
    spn = NPAD // NSUB
    mesh, cp = _mesh_and_params()

    @functools.partial(
        pl.kernel,
        mesh=mesh,
        compiler_params=cp,
        out_type=(
            jax.ShapeDtypeStruct((n_edges,), jnp.float32),
            jax.ShapeDtypeStruct((NPAD,), jnp.float32),
            jax.ShapeDtypeStruct((NPAD,), jnp.float32),
            jax.ShapeDtypeStruct((NSUB * NPAD,), jnp.float32),
            jax.ShapeDtypeStruct((NSUB * NPAD,), jnp.float32),
        ),
        scratch_types=[
            pltpu.VMEM((n_nodes,), jnp.float32),
            pltpu.VMEM((n_nodes,), jnp.float32),
            pltpu.VMEM((SBW,), jnp.int32),
            pltpu.VMEM((SBW,), jnp.int32),
            pltpu.VMEM((SBW,), jnp.float32),
            pltpu.VMEM((NPAD,), jnp.float32),
            pltpu.VMEM((spn,), jnp.float32),
        ],
    )
    def weight_kernel(src_hbm, dst_hbm, el_hbm, er_hbm,
                      w_hbm, outs0_hbm, outs1_hbm, sp0_hbm, sp1_hbm,
                      el_v, er_v, srcb, dstb, wb, s_loc, s_out):
        cid = lax.axis_index("c")
        sid = lax.axis_index("s")
        wid = sid * NCORE + cid
        is_c0 = cid == 0
        base_e = wid * epw
        pltpu.sync_copy(el_hbm, el_v)
        pltpu.sync_copy(er_hbm, er_v)
        zvec = jnp.zeros((16,), jnp.float32)

        @pl.loop(0, NPAD, step=16)
        def _(i):
            s_loc[pl.ds(i, 16)] = zvec

        @pl.loop(0, epw, step=SBW)
        def _(sb):
            pltpu.sync_copy(src_hbm.at[pl.ds(base_e + sb, SBW)], srcb)
            pltpu.sync_copy(dst_hbm.at[pl.ds(base_e + sb, SBW)], dstb)

            @pl.loop(0, SBW, step=16)
            def _(c):
                sv = srcb[pl.ds(c, 16)]
                dv = dstb[pl.ds(c, 16)]
                e = plsc.load_gather(el_v, [sv]) + plsc.load_gather(er_v, [dv])
                e = jnp.where(e > 0, e, 0.2 * e)
                w_vec = jnp.exp(e)
                wb[pl.ds(c, 16)] = w_vec
                plsc.addupdate_scatter(s_loc, [dv], w_vec)

            pltpu.sync_copy(wb, w_hbm.at[pl.ds(base_e + sb, SBW)])

        @pl.when(is_c0)
        def _():
            pltpu.sync_copy(s_loc, sp0_hbm.at[pl.ds(sid * NPAD, NPAD)])

        @pl.when(jnp.logical_not(is_c0))
        def _():
            pltpu.sync_copy(s_loc, sp1_hbm.at[pl.ds(sid * NPAD, NPAD)])
        plsc.subcore_barrier()

        def reduce_dump(sp_hbm, outs_hbm):
            for t in range(NSUB):
                pltpu.sync_copy(sp_hbm.at[pl.ds(t * NPAD + sid * spn, spn)],
                                s_loc.at[pl.ds(t * spn, spn)])

            @pl.loop(0, spn, step=16)
            def _(g):
                tot = s_loc[pl.ds(g, 16)]
                for t in range(1, NSUB):
                    tot = tot + s_loc[pl.ds(t * spn + g, 16)]
                s_out[pl.ds(g, 16)] = tot

            pltpu.sync_copy(s_out, outs_hbm.at[pl.ds(sid * spn, spn)])

        @pl.when(is_c0)
        def _():
            reduce_dump(sp0_hbm, outs0_hbm)

        @pl.when(jnp.logical_not(is_c0))
        def _():
            reduce_dump(sp1_hbm, outs1_hbm)

    return weight_kernel


def _sc_agg_kernel(n_nodes, n_edges):
    epw = n_edges // (NSUB * NCORE)
    mrs = 624
    rem = n_nodes - NSUB * mrs
    mesh, cp = _mesh_and_params()

    @functools.partial(
        pl.kernel,
        mesh=mesh,
        compiler_params=cp,
        out_type=jax.ShapeDtypeStruct((NCORE, n_nodes, 128), jnp.float32),
        scratch_types=[
            pltpu.VMEM((SBE,), jnp.int32),
            pltpu.VMEM((SBE,), jnp.int32),
            pltpu.VMEM((SBE,), jnp.float32),
            pltpu.VMEM((KBLK,), jnp.int32),
            pltpu.VMEM((KBLK,), jnp.int32),
            pltpu.VMEM((KBLK, 128), jnp.float32),
            pltpu.VMEM((KBLK, 128), jnp.float32),
            pltpu.VMEM((KBLK, 128), jnp.float32),
            pltpu.VMEM((KBLK, 128), jnp.float32),
            pltpu.VMEM_SHARED((n_nodes, 128), jnp.float32),
            pltpu.SemaphoreType.DMA,
            pltpu.SemaphoreType.DMA,
            pltpu.SemaphoreType.DMA,
            pltpu.SemaphoreType.DMA,
        ],
    )
    def agg_kernel(h_hbm, src_hbm, dst_hbm, w_hbm, z_hbm, outp_hbm,
                   srca, dsta, wa, dstb_a, dstb_b,
                   rows_a, rows_b, msg_a, msg_b, acc_sh,
                   sem_ga, sem_gb, sem_sa, sem_sb):
        cid = lax.axis_index("c")
        sid = lax.axis_index("s")
        wid = sid * NCORE + cid
        base_e = wid * epw
        npair = SBE // (2 * KBLK)
        blast = 2 * KBLK * npair

        pltpu.sync_copy(z_hbm.at[pl.ds(sid * mrs, mrs)],
                        acc_sh.at[pl.ds(sid * mrs, mrs)])

        @pl.when(sid == 0)
        def _():
            pltpu.sync_copy(z_hbm.at[pl.ds(NSUB * mrs, rem)],
                            acc_sh.at[pl.ds(NSUB * mrs, rem)])
        plsc.subcore_barrier()

        def gather_start(b, rows_x, sem_x):
            pltpu.async_copy(h_hbm.at[srca.at[pl.ds(b, KBLK)]], rows_x, sem_x)

        def gather_wait(b, rows_x, sem_x):
            pltpu.make_async_copy(
                h_hbm.at[srca.at[pl.ds(b, KBLK)]], rows_x, sem_x).wait()

        def scatter_start(msg_x, dstb_x, sem_x):
            pltpu.async_copy(msg_x, acc_sh.at[dstb_x], sem_x, add=True)

        def scatter_wait(msg_x, dstb_x, sem_x):
            pltpu.make_async_copy(msg_x, acc_sh.at[dstb_x], sem_x).wait()

        def process(b, rows_x, msg_x, dstb_x):
            for c16 in range(KBLK // 16):
                dstb_x[pl.ds(c16 * 16, 16)] = dsta[pl.ds(b + c16 * 16, 16)]
                w_vec = wa[pl.ds(b + c16 * 16, 16)]
                for jl in range(16):
                    j = c16 * 16 + jl
                    w = w_vec[jl]
                    for k in range(8):
                        msg_x[j, pl.ds(k * 16, 16)] = (
                            rows_x[j, pl.ds(k * 16, 16)] * w)

        @pl.loop(0, epw, step=SBE)
        def _(sb):
            pltpu.sync_copy(src_hbm.at[pl.ds(base_e + sb, SBE)], srca)
            pltpu.sync_copy(dst_hbm.at[pl.ds(base_e + sb, SBE)], dsta)
            pltpu.sync_copy(w_hbm.at[pl.ds(base_e + sb, SBE)], wa)
            gather_start(0, rows_a, sem_ga)

            @pl.loop(0, npair)
            def _(t):
                b0 = 2 * KBLK * t
                b1 = b0 + KBLK
                gather_start(b1, rows_b, sem_gb)

                @pl.when(t > 0)
                def _():
                    scatter_wait(msg_a, dstb_a, sem_sa)
                gather_wait(b0, rows_a, sem_ga)
                process(b0, rows_a, msg_a, dstb_a)
                scatter_start(msg_a, dstb_a, sem_sa)

                gather_start(b0 + 2 * KBLK, rows_a, sem_ga)

                @pl.when(t > 0)
                def _():
                    scatter_wait(msg_b, dstb_b, sem_sb)
                gather_wait(b1, rows_b, sem_gb)
                process(b1, rows_b, msg_b, dstb_b)
                scatter_start(msg_b, dstb_b, sem_sb)

            scatter_wait(msg_a, dstb_a, sem_sa)
            gather_wait(blast, rows_a, sem_ga)
            process(blast, rows_a, msg_a, dstb_a)
            scatter_start(msg_a, dstb_a, sem_sa)
            scatter_wait(msg_a, dstb_a, sem_sa)
            scatter_wait(msg_b, dstb_b, sem_sb)

        plsc.subcore_barrier()
        pltpu.sync_copy(acc_sh.at[pl.ds(sid * mrs, mrs)],
                        outp_hbm.at[cid, pl.ds(sid * mrs, mrs)])

        @pl.when(sid == 0)
        def _():
            pltpu.sync_copy(acc_sh.at[pl.ds(NSUB * mrs, rem)],
                            outp_hbm.at[cid, pl.ds(NSUB * mrs, rem)])

    return agg_kernel


def kernel(x, edge_index, W, a_l, a_r):
    n, d = x.shape
    e = edge_index.shape[1]
    src = edge_index[0]
    dst = edge_index[1]

    h, el2, er2 = pl.pallas_call(
        _proj_body,
        out_shape=(
            jax.ShapeDtypeStruct((n, d), jnp.float32),
            jax.ShapeDtypeStruct((n, 1), jnp.float32),
            jax.ShapeDtypeStruct((n, 1), jnp.float32),
        ),
    )(x, W, a_l.reshape(d, 1), a_r.reshape(d, 1))

    w_all, s0, s1, _, _ = _sc_weight_kernel(n, e)(
        src, dst, el2.reshape(n), er2.reshape(n))

    zacc = jnp.zeros((n, 128), jnp.float32)
    partials = _sc_agg_kernel(n, e)(h, src, dst, w_all, zacc)

    out = pl.pallas_call(
        _final_body,
        out_shape=jax.ShapeDtypeStruct((n, d), jnp.float32),
    )(partials, s0[:n].reshape(n, 1), s1[:n].reshape(n, 1), x)
    return out


# speedup vs baseline: 34.1630x; 1.0202x over previous
import dataclasses
import functools

import jax
import jax.numpy as jnp
from jax import lax
from jax.experimental import pallas as pl
from jax.experimental.pallas import tpu as pltpu
from jax.experimental.pallas import tpu_sc as plsc

KBLK = 80
SBE = 2000
SBW = 10000
NSUB = 16
NCORE = 2
NPAD = 10240


def _proj_body(x_ref, w_ref, al_ref, ar_ref, h_ref, el_ref, er_ref):
    h = jnp.dot(x_ref[...], w_ref[...], preferred_element_type=jnp.float32)
    h_ref[...] = h
    el_ref[...] = jnp.dot(h, al_ref[...], preferred_element_type=jnp.float32)
    er_ref[...] = jnp.dot(h, ar_ref[...], preferred_element_type=jnp.float32)


def _final_body(p_ref, s0_ref, s1_ref, x_ref, o_ref):
    s = s0_ref[...] + s1_ref[...] + 1e-16
    o_ref[...] = (p_ref[0] + p_ref[1]) / s + x_ref[...]


def _mesh_and_params():
    mesh = plsc.VectorSubcoreMesh(core_axis_name="c", subcore_axis_name="s")
    cp = pltpu.CompilerParams()
    if "needs_layout_passes" in pltpu.CompilerParams.__dataclass_fields__:
        cp = dataclasses.replace(cp, needs_layout_passes=False)
    return mesh, cp


def _sc_weight_kernel(n_nodes, n_edges):
    epw = n_edges // (NSUB * NCORE)
    spn = NPAD // NSUB
    mesh, cp = _mesh_and_params()

    @functools.partial(
        pl.kernel,
        mesh=mesh,
        compiler_params=cp,
        out_type=(
            jax.ShapeDtypeStruct((n_edges,), jnp.float32),
            jax.ShapeDtypeStruct((NPAD,), jnp.float32),
            jax.ShapeDtypeStruct((NPAD,), jnp.float32),
            jax.ShapeDtypeStruct((NSUB * NPAD,), jnp.float32),
            jax.ShapeDtypeStruct((NSUB * NPAD,), jnp.float32),
        ),
        scratch_types=[
            pltpu.VMEM((n_nodes,), jnp.float32),
            pltpu.VMEM((n_nodes,), jnp.float32),
            pltpu.VMEM((SBW,), jnp.int32),
            pltpu.VMEM((SBW,), jnp.int32),
            pltpu.VMEM((SBW,), jnp.float32),
            pltpu.VMEM((NPAD,), jnp.float32),
            pltpu.VMEM((spn,), jnp.float32),
        ],
    )
    def weight_kernel(src_hbm, dst_hbm, el_hbm, er_hbm,
                      w_hbm, outs0_hbm, outs1_hbm, sp0_hbm, sp1_hbm,
                      el_v, er_v, srcb, dstb, wb, s_loc, s_out):
        cid = lax.axis_index("c")
        sid = lax.axis_index("s")
        wid = sid * NCORE + cid
        is_c0 = cid == 0
        base_e = wid * epw
        pltpu.sync_copy(el_hbm, el_v)
        pltpu.sync_copy(er_hbm, er_v)
        zvec = jnp.zeros((16,), jnp.float32)

        @pl.loop(0, NPAD, step=16)
        def _(i):
            s_loc[pl.ds(i, 16)] = zvec

        @pl.loop(0, epw, step=SBW)
        def _(sb):
            pltpu.sync_copy(src_hbm.at[pl.ds(base_e + sb, SBW)], srcb)
            pltpu.sync_copy(dst_hbm.at[pl.ds(base_e + sb, SBW)], dstb)

            @pl.loop(0, SBW, step=16)
            def _(c):
                sv = srcb[pl.ds(c, 16)]
                dv = dstb[pl.ds(c, 16)]
                e = plsc.load_gather(el_v, [sv]) + plsc.load_gather(er_v, [dv])
                e = jnp.where(e > 0, e, 0.2 * e)
                w_vec = jnp.exp(e)
                wb[pl.ds(c, 16)] = w_vec
                plsc.addupdate_scatter(s_loc, [dv], w_vec)

            pltpu.sync_copy(wb, w_hbm.at[pl.ds(base_e + sb, SBW)])

        @pl.when(is_c0)
        def _():
            pltpu.sync_copy(s_loc, sp0_hbm.at[pl.ds(sid * NPAD, NPAD)])

        @pl.when(jnp.logical_not(is_c0))
        def _():
            pltpu.sync_copy(s_loc, sp1_hbm.at[pl.ds(sid * NPAD, NPAD)])
        plsc.subcore_barrier()

        def reduce_dump(sp_hbm, outs_hbm):
            for t in range(NSUB):
                pltpu.sync_copy(sp_hbm.at[pl.ds(t * NPAD + sid * spn, spn)],
                                s_loc.at[pl.ds(t * spn, spn)])

            @pl.loop(0, spn, step=16)
            def _(g):
                tot = s_loc[pl.ds(g, 16)]
                for t in range(1, NSUB):
                    tot = tot + s_loc[pl.ds(t * spn + g, 16)]
                s_out[pl.ds(g, 16)] = tot

            pltpu.sync_copy(s_out, outs_hbm.at[pl.ds(sid * spn, spn)])

        @pl.when(is_c0)
        def _():
            reduce_dump(sp0_hbm, outs0_hbm)

        @pl.when(jnp.logical_not(is_c0))
        def _():
            reduce_dump(sp1_hbm, outs1_hbm)

    return weight_kernel


def _sc_agg_kernel(n_nodes, n_edges):
    epw = n_edges // (NSUB * NCORE)
    mrs = 624
    rem = n_nodes - NSUB * mrs
    mesh, cp = _mesh_and_params()

    @functools.partial(
        pl.kernel,
        mesh=mesh,
        compiler_params=cp,
        out_type=jax.ShapeDtypeStruct((NCORE, n_nodes, 128), jnp.float32),
        scratch_types=[
            pltpu.VMEM((SBE,), jnp.int32),
            pltpu.VMEM((SBE,), jnp.int32),
            pltpu.VMEM((SBE,), jnp.float32),
            pltpu.VMEM((KBLK,), jnp.int32),
            pltpu.VMEM((KBLK,), jnp.int32),
            pltpu.VMEM((KBLK, 128), jnp.float32),
            pltpu.VMEM((KBLK, 128), jnp.float32),
            pltpu.VMEM((KBLK, 128), jnp.float32),
            pltpu.VMEM((KBLK, 128), jnp.float32),
            pltpu.VMEM_SHARED((n_nodes, 128), jnp.float32),
            pltpu.SemaphoreType.DMA,
            pltpu.SemaphoreType.DMA,
            pltpu.SemaphoreType.DMA,
            pltpu.SemaphoreType.DMA,
        ],
    )
    def agg_kernel(h_hbm, src_hbm, dst_hbm, w_hbm, z_hbm, outp_hbm,
                   srca, dsta, wa, dstb_a, dstb_b,
                   rows_a, rows_b, msg_a, msg_b, acc_sh,
                   sem_ga, sem_gb, sem_sa, sem_sb):
        cid = lax.axis_index("c")
        sid = lax.axis_index("s")
        wid = sid * NCORE + cid
        base_e = wid * epw
        npair = SBE // (2 * KBLK)
        blast = 2 * KBLK * npair

        pltpu.sync_copy(z_hbm.at[pl.ds(sid * mrs, mrs)],
                        acc_sh.at[pl.ds(sid * mrs, mrs)])

        @pl.when(sid == 0)
        def _():
            pltpu.sync_copy(z_hbm.at[pl.ds(NSUB * mrs, rem)],
                            acc_sh.at[pl.ds(NSUB * mrs, rem)])
        plsc.subcore_barrier()

        def gather_start(b, rows_x, sem_x):
            pltpu.async_copy(h_hbm.at[srca.at[pl.ds(b, KBLK)]], rows_x, sem_x)

        def gather_wait(b, rows_x, sem_x):
            pltpu.make_async_copy(
                h_hbm.at[srca.at[pl.ds(b, KBLK)]], rows_x, sem_x).wait()

        def scatter_start(msg_x, dstb_x, sem_x):
            pltpu.async_copy(msg_x, acc_sh.at[dstb_x], sem_x, add=True)

        def scatter_wait(msg_x, dstb_x, sem_x):
            pltpu.make_async_copy(msg_x, acc_sh.at[dstb_x], sem_x).wait()

        def process(b, rows_x, msg_x, dstb_x):
            for c16 in range(KBLK // 16):
                dstb_x[pl.ds(c16 * 16, 16)] = dsta[pl.ds(b + c16 * 16, 16)]
                w_vec = wa[pl.ds(b + c16 * 16, 16)]
                for jl in range(16):
                    j = c16 * 16 + jl
                    w = w_vec[jl]
                    v = rows_x[j, pl.ds(0, 16)]
                    for k in range(8):
                        m = v * w
                        if k < 7:
                            v = rows_x[j, pl.ds((k + 1) * 16, 16)]
                        msg_x[j, pl.ds(k * 16, 16)] = m

        @pl.loop(0, epw, step=SBE)
        def _(sb):
            pltpu.sync_copy(src_hbm.at[pl.ds(base_e + sb, SBE)], srca)
            pltpu.sync_copy(dst_hbm.at[pl.ds(base_e + sb, SBE)], dsta)
            pltpu.sync_copy(w_hbm.at[pl.ds(base_e + sb, SBE)], wa)
            gather_start(0, rows_a, sem_ga)

            @pl.loop(0, npair)
            def _(t):
                b0 = 2 * KBLK * t
                b1 = b0 + KBLK
                gather_start(b1, rows_b, sem_gb)

                @pl.when(t > 0)
                def _():
                    scatter_wait(msg_a, dstb_a, sem_sa)
                gather_wait(b0, rows_a, sem_ga)
                process(b0, rows_a, msg_a, dstb_a)
                scatter_start(msg_a, dstb_a, sem_sa)

                gather_start(b0 + 2 * KBLK, rows_a, sem_ga)

                @pl.when(t > 0)
                def _():
                    scatter_wait(msg_b, dstb_b, sem_sb)
                gather_wait(b1, rows_b, sem_gb)
                process(b1, rows_b, msg_b, dstb_b)
                scatter_start(msg_b, dstb_b, sem_sb)

            scatter_wait(msg_a, dstb_a, sem_sa)
            gather_wait(blast, rows_a, sem_ga)
            process(blast, rows_a, msg_a, dstb_a)
            scatter_start(msg_a, dstb_a, sem_sa)
            scatter_wait(msg_a, dstb_a, sem_sa)
            scatter_wait(msg_b, dstb_b, sem_sb)

        plsc.subcore_barrier()
        pltpu.sync_copy(acc_sh.at[pl.ds(sid * mrs, mrs)],
                        outp_hbm.at[cid, pl.ds(sid * mrs, mrs)])

        @pl.when(sid == 0)
        def _():
            pltpu.sync_copy(acc_sh.at[pl.ds(NSUB * mrs, rem)],
                            outp_hbm.at[cid, pl.ds(NSUB * mrs, rem)])

    return agg_kernel


def kernel(x, edge_index, W, a_l, a_r):
    n, d = x.shape
    e = edge_index.shape[1]
    src = edge_index[0]
    dst = edge_index[1]

    h, el2, er2 = pl.pallas_call(
        _proj_body,
        out_shape=(
            jax.ShapeDtypeStruct((n, d), jnp.float32),
            jax.ShapeDtypeStruct((n, 1), jnp.float32),
            jax.ShapeDtypeStruct((n, 1), jnp.float32),
        ),
    )(x, W, a_l.reshape(d, 1), a_r.reshape(d, 1))

    w_all, s0, s1, _, _ = _sc_weight_kernel(n, e)(
        src, dst, el2.reshape(n), er2.reshape(n))

    zacc = jnp.zeros((n, 128), jnp.float32)
    partials = _sc_agg_kernel(n, e)(h, src, dst, w_all, zacc)

    out = pl.pallas_call(
        _final_body,
        out_shape=jax.ShapeDtypeStruct((n, d), jnp.float32),
    )(partials, s0[:n].reshape(n, 1), s1[:n].reshape(n, 1), x)
    return out
